# baseline (speedup 1.0000x reference)
import functools

import jax
import jax.numpy as jnp
from jax import lax
from jax.experimental import pallas as pl
from jax.experimental.pallas import tpu as pltpu
from jax.experimental.pallas import tpu_sc as plsc

B, N, D = 16, 2048, 1024
NC, NS = 2, 16
NW = NC * NS
DW = D // 2
CH = 64
LANES = 16
JV = DW // LANES

NT = 2048
SC_N = N - NT
NCHUNK = SC_N // CH
CHT = 256


def _sc_body(x_hbm, out_hbm, buf, acc, sem0, sem1):
    wid = lax.axis_index("c") * NS + lax.axis_index("s")
    b = wid // 2
    c0 = (wid % 2) * DW

    sems = (sem0, sem1)

    def start(i, slot):
        pltpu.async_copy(
            x_hbm.at[b, pl.ds(NT + i * CH, CH), pl.ds(c0, DW)],
            buf.at[slot], sems[slot])

    def wait(slot):
        pltpu.make_async_copy(
            x_hbm.at[b, pl.ds(0, CH), pl.ds(c0, DW)],
            buf.at[slot], sems[slot]).wait()

    def accumulate(slot, carry):
        def row_body(r, carry):
            vals = [buf[slot, r, pl.ds(j * LANES, LANES)] for j in range(JV)]
            return tuple(c + v for c, v in zip(carry, vals))

        return lax.fori_loop(0, CH, row_body, carry)

    start(0, 0)
    zeros = tuple(jnp.zeros((LANES,), jnp.float32) for _ in range(JV))

    @pl.loop(0, NCHUNK, step=2, init_carry=zeros)
    def _chunk(g, carry):
        start(g + 1, 1)
        wait(0)
        carry = accumulate(0, carry)

        @pl.when(g + 2 < NCHUNK)
        def _():
            start(g + 2, 0)

        wait(1)
        return accumulate(1, carry)

    for j in range(JV):
        acc[pl.ds(j * LANES, LANES)] = _chunk[j]

    pltpu.sync_copy(acc, out_hbm.at[b, pl.ds(c0, DW)])


_mesh = plsc.VectorSubcoreMesh(core_axis_name="c", subcore_axis_name="s")

_sum_sc = functools.partial(
    pl.kernel,
    out_type=jax.ShapeDtypeStruct((B, D), jnp.float32),
    mesh=_mesh,
    scratch_types=[
        pltpu.VMEM((2, CH, DW), jnp.float32),
        pltpu.VMEM((DW,), jnp.float32),
        pltpu.SemaphoreType.DMA,
        pltpu.SemaphoreType.DMA,
    ],
)(_sc_body)


def _tc_body(x0, x1, x2, x3, out_ref):
    k = pl.program_id(1)
    partial = jnp.concatenate(
        [jnp.sum(x[0], axis=0, keepdims=True) for x in (x0, x1, x2, x3)],
        axis=-1)[None]

    @pl.when(k == 0)
    def _():
        out_ref[...] = partial

    @pl.when(k > 0)
    def _():
        out_ref[...] += partial


_sum_tc = pl.pallas_call(
    _tc_body,
    grid=(B, NT // CHT),
    in_specs=[pl.BlockSpec((1, CHT, D // 4), lambda b, k, ci=ci: (b, k, ci))
              for ci in range(4)],
    out_specs=pl.BlockSpec((1, 1, D), lambda b, k: (b, 0, 0)),
    out_shape=jax.ShapeDtypeStruct((B, 1, D), jnp.float32),
    compiler_params=pltpu.CompilerParams(
        dimension_semantics=("parallel", "arbitrary")),
)


@jax.jit
def kernel(inputs):
    tc_part = _sum_tc(inputs, inputs, inputs, inputs)
    return tc_part[:, 0, :]


# speedup vs baseline: 2.1199x; 2.1199x over previous
import functools

import jax
import jax.numpy as jnp
from jax import lax
from jax.experimental import pallas as pl
from jax.experimental.pallas import tpu as pltpu
from jax.experimental.pallas import tpu_sc as plsc

B, N, D = 16, 2048, 1024
NC, NS = 2, 16
NW = NC * NS
DW = D // 2
CH = 64
LANES = 16
JV = DW // LANES

NT = 2048
SC_N = N - NT
NCHUNK = SC_N // CH
CHT = 256


def _sc_body(x_hbm, out_hbm, buf, acc, sem0, sem1):
    wid = lax.axis_index("c") * NS + lax.axis_index("s")
    b = wid // 2
    c0 = (wid % 2) * DW

    sems = (sem0, sem1)

    def start(i, slot):
        pltpu.async_copy(
            x_hbm.at[b, pl.ds(NT + i * CH, CH), pl.ds(c0, DW)],
            buf.at[slot], sems[slot])

    def wait(slot):
        pltpu.make_async_copy(
            x_hbm.at[b, pl.ds(0, CH), pl.ds(c0, DW)],
            buf.at[slot], sems[slot]).wait()

    def accumulate(slot, carry):
        def row_body(r, carry):
            vals = [buf[slot, r, pl.ds(j * LANES, LANES)] for j in range(JV)]
            return tuple(c + v for c, v in zip(carry, vals))

        return lax.fori_loop(0, CH, row_body, carry)

    start(0, 0)
    zeros = tuple(jnp.zeros((LANES,), jnp.float32) for _ in range(JV))

    @pl.loop(0, NCHUNK, step=2, init_carry=zeros)
    def _chunk(g, carry):
        start(g + 1, 1)
        wait(0)
        carry = accumulate(0, carry)

        @pl.when(g + 2 < NCHUNK)
        def _():
            start(g + 2, 0)

        wait(1)
        return accumulate(1, carry)

    for j in range(JV):
        acc[pl.ds(j * LANES, LANES)] = _chunk[j]

    pltpu.sync_copy(acc, out_hbm.at[b, pl.ds(c0, DW)])


_mesh = plsc.VectorSubcoreMesh(core_axis_name="c", subcore_axis_name="s")

_sum_sc = functools.partial(
    pl.kernel,
    out_type=jax.ShapeDtypeStruct((B, D), jnp.float32),
    mesh=_mesh,
    scratch_types=[
        pltpu.VMEM((2, CH, DW), jnp.float32),
        pltpu.VMEM((DW,), jnp.float32),
        pltpu.SemaphoreType.DMA,
        pltpu.SemaphoreType.DMA,
    ],
)(_sc_body)


RB = 8
CHK = 128
NCH_TC = (B * NT) // (CHK * 16) * 16


def _tc_body(x_hbm, out_ref, buf, sem):
    nsteps = B * (NT // CHK)
    kpb = NT // CHK

    out_ref[...] = jnp.zeros((B, D), jnp.float32)

    def start_dma(c, slot):
        b = c // kpb
        k = c % kpb
        pltpu.async_copy(
            x_hbm.at[b, pl.ds(k * CHK, CHK), :], buf.at[slot], sem.at[slot])

    for c in range(RB):
        start_dma(c, c)

    def body(c, _):
        slot = lax.rem(c, RB)
        pltpu.make_async_copy(
            x_hbm.at[0, pl.ds(0, CHK), :], buf.at[slot], sem.at[slot]).wait()
        b = c // kpb
        out_ref[pl.ds(b, 1), :] += jnp.sum(buf[slot], axis=0, keepdims=True)

        nc = c + RB

        @pl.when(nc < nsteps)
        def _():
            nb = nc // kpb
            nk = lax.rem(nc, kpb)
            pltpu.async_copy(
                x_hbm.at[nb, pl.ds(nk * CHK, CHK), :],
                buf.at[slot], sem.at[slot])

        return 0

    lax.fori_loop(0, nsteps, body, 0)


_sum_tc = pl.pallas_call(
    _tc_body,
    in_specs=[pl.BlockSpec(memory_space=pltpu.MemorySpace.HBM)],
    out_specs=pl.BlockSpec(memory_space=pltpu.MemorySpace.VMEM),
    out_shape=jax.ShapeDtypeStruct((B, D), jnp.float32),
    scratch_shapes=[
        pltpu.VMEM((RB, CHK, D), jnp.float32),
        pltpu.SemaphoreType.DMA((RB,)),
    ],
)


@jax.jit
def kernel(inputs):
    return _sum_tc(inputs)
